# baseline (speedup 1.0000x reference)
import functools

import jax
import jax.numpy as jnp
from jax import lax
from jax.experimental import pallas as pl
from jax.experimental.pallas import tpu as pltpu
from jax.experimental.pallas import tpu_sc as plsc

N = 10000
E = 320000
H = 4
F = 32
HF = H * F
ROW = HF + 16
ALPHA = 0.2

NC = 2
NS = 16
NW = NC * NS
EPW = E // NW
CHUNK = 80
NCHUNK = EPW // CHUNK
GROUPS = CHUNK // 16
NPAD = 10240
ZROWS = 128
ROWS_PER_TILE = NPAD // NS


def _node_body(h_ref, w_ref, ar_ref, ad_ref, tab_ref, sd_ref):
    w = w_ref[...]
    hb = h_ref[...]
    wh = jnp.dot(hb, w, preferred_element_type=jnp.float32)
    wm_s = w * ar_ref[...]
    wm_d = w * ad_ref[...]
    cols = []
    for hh in range(H):
        cols.append(jnp.sum(wm_s[:, hh * F:(hh + 1) * F], axis=1, keepdims=True))
    for hh in range(H):
        cols.append(jnp.sum(wm_d[:, hh * F:(hh + 1) * F], axis=1, keepdims=True))
    csd = jnp.concatenate(cols + [jnp.zeros((HF, 120), jnp.float32)], axis=1)
    s = jnp.dot(hb, csd, preferred_element_type=jnp.float32)
    bn = hb.shape[0]
    tab_ref[...] = jnp.concatenate(
        [wh, s[:, 0:4], jnp.zeros((bn, 12), jnp.float32)], axis=1)
    sd_ref[...] = jnp.concatenate(
        [s[:, 4:8], jnp.zeros((bn, 12), jnp.float32)], axis=1)


def _node_kernel(h, W, ar, ad):
    BN = 1000
    grid = (N // BN,)
    return pl.pallas_call(
        _node_body,
        grid=grid,
        in_specs=[
            pl.BlockSpec((BN, HF), lambda i: (i, 0)),
            pl.BlockSpec((HF, HF), lambda i: (0, 0)),
            pl.BlockSpec((1, HF), lambda i: (0, 0)),
            pl.BlockSpec((1, HF), lambda i: (0, 0)),
        ],
        out_specs=[
            pl.BlockSpec((BN, ROW), lambda i: (i, 0)),
            pl.BlockSpec((BN, 16), lambda i: (i, 0)),
        ],
        out_shape=[
            jax.ShapeDtypeStruct((N, ROW), jnp.float32),
            jax.ShapeDtypeStruct((N, 16), jnp.float32),
        ],
    )(h, W, ar, ad)


def _edge_body(eft_ref, we_ref, ae_ref, out_ref):
    we = we_ref[...]
    wm = we * ae_ref[...]
    cols = []
    for hh in range(H):
        cols.append(jnp.sum(wm[:, hh * F:(hh + 1) * F], axis=1, keepdims=True))
    ce = jnp.concatenate(cols, axis=1)
    out_ref[...] = lax.dot_general(
        ce, eft_ref[...], (((0,), (0,)), ((), ())),
        preferred_element_type=jnp.float32)


def _edge_kernel(eft, W_e, ae):
    BE = 6400
    EF = eft.shape[0]
    grid = (E // BE,)
    return pl.pallas_call(
        _edge_body,
        grid=grid,
        in_specs=[
            pl.BlockSpec((EF, BE), lambda i: (0, i)),
            pl.BlockSpec((EF, HF), lambda i: (0, 0)),
            pl.BlockSpec((1, HF), lambda i: (0, 0)),
        ],
        out_specs=pl.BlockSpec((H, BE), lambda i: (0, i)),
        out_shape=jax.ShapeDtypeStruct((H, E), jnp.float32),
    )(eft, W_e, ae)


def _sc_body(tab_hbm, sdst_hbm, se_hbm, ei_hbm, out_hbm,
             stage, outb, sdstb, seb, sidx, didx, dsidx, exbuf,
             acc, sem_g, sem_d, sem_sc):
    c = lax.axis_index("c")
    s = lax.axis_index("s")
    wid = c * NS + s
    zeros16 = jnp.zeros((16,), jnp.float32)

    def zrow(r, _):
        for j in range(ROW // 16):
            outb[r, pl.ds(j * 16, 16)] = zeros16
        return _
    lax.fori_loop(0, CHUNK, zrow, None)
    row0 = s * ROWS_PER_TILE
    for k in range(ROWS_PER_TILE // CHUNK):
        pltpu.sync_copy(outb, acc.at[pl.ds(row0 + k * CHUNK, CHUNK)])
    plsc.subcore_barrier()

    iota = lax.iota(jnp.int32, 16)
    rid0 = lax.shift_right_logical(iota, 2)
    cid = lax.bitwise_and(iota, 3)

    def small(p, ci):
        base = wid * EPW + ci * CHUNK
        pltpu.async_copy(ei_hbm.at[pl.ds(base, CHUNK)], sidx[p], sem_d[p])
        pltpu.async_copy(ei_hbm.at[pl.ds(E + base, CHUNK)], didx[p],
                         sem_d[p])
        pltpu.async_copy(se_hbm.at[:, pl.ds(base, CHUNK)], seb[p], sem_d[p])

    def gathers(p, ci):
        base = wid * EPW + ci * CHUNK
        pltpu.make_async_copy(ei_hbm.at[pl.ds(base, CHUNK)], sidx[p],
                              sem_d[p]).wait()
        pltpu.make_async_copy(ei_hbm.at[pl.ds(E + base, CHUNK)], didx[p],
                              sem_d[p]).wait()
        pltpu.make_async_copy(se_hbm.at[:, pl.ds(base, CHUNK)], seb[p],
                              sem_d[p]).wait()
        pltpu.async_copy(tab_hbm.at[sidx[p]], stage[p], sem_g[p])
        pltpu.async_copy(sdst_hbm.at[didx[p]], sdstb[p], sem_g[p])

    def wait_gather(p):
        pltpu.make_async_copy(tab_hbm.at[sidx[p]], stage[p], sem_g[p]).wait()
        pltpu.make_async_copy(sdst_hbm.at[didx[p]], sdstb[p],
                              sem_g[p]).wait()

    def wait_scatter():
        pltpu.make_async_copy(outb, acc.at[dsidx], sem_sc).wait()

    def compute(p):
        stg, sdb, seb_p = stage[p], sdstb[p], seb[p]
        @plsc.parallel_loop(0, CHUNK // 4, 1, unroll=2)
        def _(v):
            rid = rid0 + (4 * v)
            s1 = plsc.load_gather(stg, [rid, cid + HF])
            s2 = plsc.load_gather(sdb, [rid, cid])
            se = plsc.load_gather(seb_p, [cid, rid])
            sc = s1 + s2 + se
            sc = jnp.maximum(sc, ALPHA * sc)
            ex = jnp.exp(sc)
            exbuf[pl.ds(16 * v, 16)] = ex
            plsc.store_scatter(outb, [rid, cid + HF], ex)

        @plsc.parallel_loop(0, CHUNK // 4, 1, unroll=2)
        def _(q):
            f = exbuf[pl.ds(16 * q, 16)]
            for r in range(4):
                e = 4 * q + r
                for hh in range(H):
                    fs = f[4 * r + hh]
                    for j in range(2):
                        sl = pl.ds(hh * F + j * 16, 16)
                        outb[e, sl] = stg[e, sl] * fs

    def scatter(p):
        for t in range(CHUNK // 16):
            dsidx[pl.ds(16 * t, 16)] = didx[p][pl.ds(16 * t, 16)]
        pltpu.async_copy(outb, acc.at[dsidx], sem_sc, add=True)

    small(0, 0)
    small(1, 1)
    gathers(0, 0)
    gathers(1, 1)
    wait_gather(0)
    compute(0)
    scatter(0)
    small(0, 2)

    def pair(k, _):
        i0 = 2 * k
        gathers(0, i0 + 2)
        wait_gather(1)
        wait_scatter()
        compute(1)
        scatter(1)
        small(1, i0 + 3)
        gathers(1, i0 + 3)
        wait_gather(0)
        wait_scatter()
        compute(0)
        scatter(0)
        small(0, i0 + 4)
        return _
    lax.fori_loop(0, (NCHUNK - 3) // 2, pair, None)

    gathers(0, 124)
    wait_gather(1)
    wait_scatter()
    compute(1)
    scatter(1)
    wait_gather(0)
    wait_scatter()
    compute(0)
    scatter(0)
    wait_scatter()

    plsc.subcore_barrier()
    for k in range(ROWS_PER_TILE // CHUNK):
        r = row0 + k * CHUNK
        pltpu.sync_copy(acc.at[pl.ds(r, CHUNK)], outb)
        pltpu.sync_copy(outb, out_hbm.at[c, pl.ds(r, CHUNK)])


def _sc_sweep(tab, sdst_tab, seT, ei_flat):
    mesh = plsc.VectorSubcoreMesh(
        core_axis_name="c", subcore_axis_name="s",
        num_cores=NC, num_subcores=NS)
    fn = pl.kernel(
        _sc_body,
        out_type=jax.ShapeDtypeStruct((NC, NPAD, ROW), jnp.float32),
        mesh=mesh,
        compiler_params=pltpu.CompilerParams(
            use_tc_tiling_on_sc=False, needs_layout_passes=False),
        scratch_types=[
            (pltpu.VMEM((CHUNK, ROW), jnp.float32),) * 2,
            pltpu.VMEM((CHUNK, ROW), jnp.float32),
            (pltpu.VMEM((CHUNK, 16), jnp.float32),) * 2,
            (pltpu.VMEM((H, CHUNK), jnp.float32),) * 2,
            (pltpu.VMEM((CHUNK,), jnp.int32),) * 2,
            (pltpu.VMEM((CHUNK,), jnp.int32),) * 2,
            pltpu.VMEM((CHUNK,), jnp.int32),
            pltpu.VMEM((CHUNK * H,), jnp.float32),
            pltpu.VMEM_SHARED((NPAD, ROW), jnp.float32),
            (pltpu.SemaphoreType.DMA,) * 2,
            (pltpu.SemaphoreType.DMA,) * 2,
            pltpu.SemaphoreType.DMA,
        ],
    )
    return fn(tab, sdst_tab, seT, ei_flat)


def _final_body(p_ref, out_ref):
    x = p_ref[0, :, :] + p_ref[1, :, :]
    den = x[:, HF:HF + H]
    rden = 1.0 / (den + 1e-16)
    parts = []
    for hh in range(H):
        parts.append(x[:, hh * F:(hh + 1) * F] * rden[:, hh:hh + 1])
    z = jnp.concatenate(parts, axis=1)
    out_ref[...] = jnp.where(z > 0, z, jnp.exp(jnp.minimum(z, 0.0)) - 1.0)


def _final_kernel(partials):
    BN = 1000
    grid = (N // BN,)
    return pl.pallas_call(
        _final_body,
        grid=grid,
        in_specs=[pl.BlockSpec((NC, BN, ROW), lambda i: (0, i, 0))],
        out_specs=pl.BlockSpec((BN, HF), lambda i: (i, 0)),
        out_shape=jax.ShapeDtypeStruct((N, HF), jnp.float32),
    )(partials)


@jax.jit
def kernel(h, edge_features, edge_index, W, W_e, a_src, a_dst, a_edge):
    ar = a_src.reshape(1, HF)
    ad = a_dst.reshape(1, HF)
    ae = a_edge.reshape(1, HF)
    tab, sdst_tab = _node_kernel(h, W, ar, ad)
    seT = _edge_kernel(edge_features.T, W_e, ae)
    partials = _sc_sweep(tab, sdst_tab, seT, edge_index.reshape(2 * E))
    return _final_kernel(partials)


# speedup vs baseline: 1.0480x; 1.0480x over previous
import functools

import jax
import jax.numpy as jnp
from jax import lax
from jax.experimental import pallas as pl
from jax.experimental.pallas import tpu as pltpu
from jax.experimental.pallas import tpu_sc as plsc

N = 10000
E = 320000
H = 4
F = 32
HF = H * F
ROW = HF + 16
ALPHA = 0.2

NC = 2
NS = 16
NW = NC * NS
EPW = E // NW
CHUNK = 80
NCHUNK = EPW // CHUNK
GROUPS = CHUNK // 16
NPAD = 10240
ZROWS = 128
ROWS_PER_TILE = NPAD // NS


def _node_body(h_ref, w_ref, ar_ref, ad_ref, tab_ref, sd_ref):
    w = w_ref[...]
    hb = h_ref[...]
    wh = jnp.dot(hb, w, preferred_element_type=jnp.float32)
    wm_s = w * ar_ref[...]
    wm_d = w * ad_ref[...]
    cols = []
    for hh in range(H):
        cols.append(jnp.sum(wm_s[:, hh * F:(hh + 1) * F], axis=1, keepdims=True))
    for hh in range(H):
        cols.append(jnp.sum(wm_d[:, hh * F:(hh + 1) * F], axis=1, keepdims=True))
    csd = jnp.concatenate(cols + [jnp.zeros((HF, 120), jnp.float32)], axis=1)
    s = jnp.dot(hb, csd, preferred_element_type=jnp.float32)
    bn = hb.shape[0]
    tab_ref[...] = jnp.concatenate(
        [wh, s[:, 0:4], jnp.zeros((bn, 12), jnp.float32)], axis=1)
    sd_ref[...] = jnp.concatenate(
        [s[:, 4:8], jnp.zeros((bn, 12), jnp.float32)], axis=1)


def _node_kernel(h, W, ar, ad):
    BN = 1000
    grid = (N // BN,)
    return pl.pallas_call(
        _node_body,
        grid=grid,
        in_specs=[
            pl.BlockSpec((BN, HF), lambda i: (i, 0)),
            pl.BlockSpec((HF, HF), lambda i: (0, 0)),
            pl.BlockSpec((1, HF), lambda i: (0, 0)),
            pl.BlockSpec((1, HF), lambda i: (0, 0)),
        ],
        out_specs=[
            pl.BlockSpec((BN, ROW), lambda i: (i, 0)),
            pl.BlockSpec((BN, 16), lambda i: (i, 0)),
        ],
        out_shape=[
            jax.ShapeDtypeStruct((N, ROW), jnp.float32),
            jax.ShapeDtypeStruct((N, 16), jnp.float32),
        ],
    )(h, W, ar, ad)


def _edge_body(eft_ref, we_ref, ae_ref, out_ref):
    we = we_ref[...]
    wm = we * ae_ref[...]
    cols = []
    for hh in range(H):
        cols.append(jnp.sum(wm[:, hh * F:(hh + 1) * F], axis=1, keepdims=True))
    ce = jnp.concatenate(cols, axis=1)
    out_ref[...] = lax.dot_general(
        ce, eft_ref[...], (((0,), (0,)), ((), ())),
        preferred_element_type=jnp.float32)


def _edge_kernel(eft, W_e, ae):
    BE = 12800
    EF = eft.shape[0]
    grid = (E // BE,)
    return pl.pallas_call(
        _edge_body,
        grid=grid,
        in_specs=[
            pl.BlockSpec((EF, BE), lambda i: (0, i)),
            pl.BlockSpec((EF, HF), lambda i: (0, 0)),
            pl.BlockSpec((1, HF), lambda i: (0, 0)),
        ],
        out_specs=pl.BlockSpec((H, BE), lambda i: (0, i)),
        out_shape=jax.ShapeDtypeStruct((H, E), jnp.float32),
    )(eft, W_e, ae)


def _sc_body(tab_hbm, sdst_hbm, se_hbm, ei_hbm, out_hbm,
             stage, outb, sdstb, seb, sidx, didx, dsidx, exbuf,
             acc, sem_g, sem_d, sem_sc):
    c = lax.axis_index("c")
    s = lax.axis_index("s")
    wid = c * NS + s
    zeros16 = jnp.zeros((16,), jnp.float32)

    def zrow(r, _):
        for j in range(ROW // 16):
            outb[r, pl.ds(j * 16, 16)] = zeros16
        return _
    lax.fori_loop(0, CHUNK, zrow, None)
    row0 = s * ROWS_PER_TILE
    for k in range(ROWS_PER_TILE // CHUNK):
        pltpu.sync_copy(outb, acc.at[pl.ds(row0 + k * CHUNK, CHUNK)])
    plsc.subcore_barrier()

    iota = lax.iota(jnp.int32, 16)
    rid0 = lax.shift_right_logical(iota, 2)
    cid = lax.bitwise_and(iota, 3)

    def small(p, ci):
        base = wid * EPW + ci * CHUNK
        pltpu.async_copy(ei_hbm.at[pl.ds(base, CHUNK)], sidx[p], sem_d[p])
        pltpu.async_copy(ei_hbm.at[pl.ds(E + base, CHUNK)], didx[p],
                         sem_d[p])
        pltpu.async_copy(se_hbm.at[:, pl.ds(base, CHUNK)], seb[p], sem_d[p])

    def gathers(p, ci):
        base = wid * EPW + ci * CHUNK
        pltpu.make_async_copy(ei_hbm.at[pl.ds(base, CHUNK)], sidx[p],
                              sem_d[p]).wait()
        pltpu.make_async_copy(ei_hbm.at[pl.ds(E + base, CHUNK)], didx[p],
                              sem_d[p]).wait()
        pltpu.make_async_copy(se_hbm.at[:, pl.ds(base, CHUNK)], seb[p],
                              sem_d[p]).wait()
        pltpu.async_copy(tab_hbm.at[sidx[p]], stage[p], sem_g[p])
        pltpu.async_copy(sdst_hbm.at[didx[p]], sdstb[p], sem_g[p])

    def wait_gather(p):
        pltpu.make_async_copy(tab_hbm.at[sidx[p]], stage[p], sem_g[p]).wait()
        pltpu.make_async_copy(sdst_hbm.at[didx[p]], sdstb[p],
                              sem_g[p]).wait()

    def wait_scatter():
        pltpu.make_async_copy(outb, acc.at[dsidx], sem_sc).wait()

    def compute(p):
        stg, sdb, seb_p = stage[p], sdstb[p], seb[p]
        @plsc.parallel_loop(0, CHUNK // 4, 1, unroll=2)
        def _(v):
            rid = rid0 + (4 * v)
            s1 = plsc.load_gather(stg, [rid, cid + HF])
            s2 = plsc.load_gather(sdb, [rid, cid])
            se = plsc.load_gather(seb_p, [cid, rid])
            sc = s1 + s2 + se
            sc = jnp.maximum(sc, ALPHA * sc)
            ex = jnp.exp(sc)
            exbuf[pl.ds(16 * v, 16)] = ex
            plsc.store_scatter(outb, [rid, cid + HF], ex)

        @plsc.parallel_loop(0, CHUNK // 4, 1, unroll=4)
        def _(q):
            f = exbuf[pl.ds(16 * q, 16)]
            for r in range(4):
                e = 4 * q + r
                for hh in range(H):
                    fs = f[4 * r + hh]
                    for j in range(2):
                        sl = pl.ds(hh * F + j * 16, 16)
                        outb[e, sl] = stg[e, sl] * fs

    def scatter(p):
        for t in range(CHUNK // 16):
            dsidx[pl.ds(16 * t, 16)] = didx[p][pl.ds(16 * t, 16)]
        pltpu.async_copy(outb, acc.at[dsidx], sem_sc, add=True)

    small(0, 0)
    small(1, 1)
    gathers(0, 0)
    gathers(1, 1)
    wait_gather(0)
    compute(0)
    scatter(0)
    small(0, 2)

    def pair(k, _):
        i0 = 2 * k
        gathers(0, i0 + 2)
        wait_gather(1)
        wait_scatter()
        compute(1)
        scatter(1)
        small(1, i0 + 3)
        gathers(1, i0 + 3)
        wait_gather(0)
        wait_scatter()
        compute(0)
        scatter(0)
        small(0, i0 + 4)
        return _
    lax.fori_loop(0, (NCHUNK - 3) // 2, pair, None)

    gathers(0, 124)
    wait_gather(1)
    wait_scatter()
    compute(1)
    scatter(1)
    wait_gather(0)
    wait_scatter()
    compute(0)
    scatter(0)
    wait_scatter()

    plsc.subcore_barrier()
    for k in range(ROWS_PER_TILE // CHUNK):
        r = row0 + k * CHUNK
        pltpu.sync_copy(acc.at[pl.ds(r, CHUNK)], outb)
        pltpu.sync_copy(outb, out_hbm.at[c, pl.ds(r, CHUNK)])


def _sc_sweep(tab, sdst_tab, seT, ei_flat):
    mesh = plsc.VectorSubcoreMesh(
        core_axis_name="c", subcore_axis_name="s",
        num_cores=NC, num_subcores=NS)
    fn = pl.kernel(
        _sc_body,
        out_type=jax.ShapeDtypeStruct((NC, NPAD, ROW), jnp.float32),
        mesh=mesh,
        compiler_params=pltpu.CompilerParams(
            use_tc_tiling_on_sc=False, needs_layout_passes=False),
        scratch_types=[
            (pltpu.VMEM((CHUNK, ROW), jnp.float32),) * 2,
            pltpu.VMEM((CHUNK, ROW), jnp.float32),
            (pltpu.VMEM((CHUNK, 16), jnp.float32),) * 2,
            (pltpu.VMEM((H, CHUNK), jnp.float32),) * 2,
            (pltpu.VMEM((CHUNK,), jnp.int32),) * 2,
            (pltpu.VMEM((CHUNK,), jnp.int32),) * 2,
            pltpu.VMEM((CHUNK,), jnp.int32),
            pltpu.VMEM((CHUNK * H,), jnp.float32),
            pltpu.VMEM_SHARED((NPAD, ROW), jnp.float32),
            (pltpu.SemaphoreType.DMA,) * 2,
            (pltpu.SemaphoreType.DMA,) * 2,
            pltpu.SemaphoreType.DMA,
        ],
    )
    return fn(tab, sdst_tab, seT, ei_flat)


def _final_body(p_ref, out_ref):
    x = p_ref[0, :, :] + p_ref[1, :, :]
    den = x[:, HF:HF + H]
    rden = 1.0 / (den + 1e-16)
    parts = []
    for hh in range(H):
        parts.append(x[:, hh * F:(hh + 1) * F] * rden[:, hh:hh + 1])
    z = jnp.concatenate(parts, axis=1)
    out_ref[...] = jnp.where(z > 0, z, jnp.exp(jnp.minimum(z, 0.0)) - 1.0)


def _final_kernel(partials):
    BN = 1000
    grid = (N // BN,)
    return pl.pallas_call(
        _final_body,
        grid=grid,
        in_specs=[pl.BlockSpec((NC, BN, ROW), lambda i: (0, i, 0))],
        out_specs=pl.BlockSpec((BN, HF), lambda i: (i, 0)),
        out_shape=jax.ShapeDtypeStruct((N, HF), jnp.float32),
    )(partials)


@jax.jit
def kernel(h, edge_features, edge_index, W, W_e, a_src, a_dst, a_edge):
    ar = a_src.reshape(1, HF)
    ad = a_dst.reshape(1, HF)
    ae = a_edge.reshape(1, HF)
    tab, sdst_tab = _node_kernel(h, W, ar, ad)
    seT = _edge_kernel(edge_features.T, W_e, ae)
    partials = _sc_sweep(tab, sdst_tab, seT, edge_index.reshape(2 * E))
    return _final_kernel(partials)


# speedup vs baseline: 1.1244x; 1.0729x over previous
import functools

import jax
import jax.numpy as jnp
from jax import lax
from jax.experimental import pallas as pl
from jax.experimental.pallas import tpu as pltpu
from jax.experimental.pallas import tpu_sc as plsc

N = 10000
E = 320000
H = 4
F = 32
HF = H * F
ROW = HF + 16
ALPHA = 0.2

NC = 2
NS = 16
NW = NC * NS
EPW = E // NW
CHUNK = 80
NCHUNK = EPW // CHUNK
GROUPS = CHUNK // 16
NPAD = 10240
ZROWS = 128
ROWS_PER_TILE = NPAD // NS


def _node_body(h_ref, w_ref, ar_ref, ad_ref, tab_ref, sd_ref):
    w = w_ref[...]
    hb = h_ref[...]
    wh = jnp.dot(hb, w, preferred_element_type=jnp.float32)
    wm_s = w * ar_ref[...]
    wm_d = w * ad_ref[...]
    cols = []
    for hh in range(H):
        cols.append(jnp.sum(wm_s[:, hh * F:(hh + 1) * F], axis=1, keepdims=True))
    for hh in range(H):
        cols.append(jnp.sum(wm_d[:, hh * F:(hh + 1) * F], axis=1, keepdims=True))
    csd = jnp.concatenate(cols + [jnp.zeros((HF, 120), jnp.float32)], axis=1)
    s = jnp.dot(hb, csd, preferred_element_type=jnp.float32)
    bn = hb.shape[0]
    tab_ref[...] = jnp.concatenate(
        [wh, s[:, 0:4], jnp.zeros((bn, 12), jnp.float32)], axis=1)
    sd_ref[...] = jnp.concatenate(
        [s[:, 4:8], jnp.zeros((bn, 12), jnp.float32)], axis=1)


def _node_kernel(h, W, ar, ad):
    BN = 1000
    grid = (N // BN,)
    return pl.pallas_call(
        _node_body,
        grid=grid,
        in_specs=[
            pl.BlockSpec((BN, HF), lambda i: (i, 0)),
            pl.BlockSpec((HF, HF), lambda i: (0, 0)),
            pl.BlockSpec((1, HF), lambda i: (0, 0)),
            pl.BlockSpec((1, HF), lambda i: (0, 0)),
        ],
        out_specs=[
            pl.BlockSpec((BN, ROW), lambda i: (i, 0)),
            pl.BlockSpec((BN, 16), lambda i: (i, 0)),
        ],
        out_shape=[
            jax.ShapeDtypeStruct((N, ROW), jnp.float32),
            jax.ShapeDtypeStruct((N, 16), jnp.float32),
        ],
    )(h, W, ar, ad)


def _edge_body(eft_ref, we_ref, ae_ref, out_ref):
    we = we_ref[...]
    wm = we * ae_ref[...]
    cols = []
    for hh in range(H):
        cols.append(jnp.sum(wm[:, hh * F:(hh + 1) * F], axis=1, keepdims=True))
    ce = jnp.concatenate(cols, axis=1)
    out_ref[...] = lax.dot_general(
        ce, eft_ref[...], (((0,), (0,)), ((), ())),
        preferred_element_type=jnp.float32)


def _edge_kernel(eft, W_e, ae):
    BE = 12800
    EF = eft.shape[0]
    grid = (E // BE,)
    return pl.pallas_call(
        _edge_body,
        grid=grid,
        in_specs=[
            pl.BlockSpec((EF, BE), lambda i: (0, i)),
            pl.BlockSpec((EF, HF), lambda i: (0, 0)),
            pl.BlockSpec((1, HF), lambda i: (0, 0)),
        ],
        out_specs=pl.BlockSpec((H, BE), lambda i: (0, i)),
        out_shape=jax.ShapeDtypeStruct((H, E), jnp.float32),
    )(eft, W_e, ae)


def _sc_body(tab_hbm, sdst_hbm, se_hbm, ei_hbm, out_hbm,
             stage, outb, sdstb, seb, sidx, didx, dsidx, exbuf,
             acc, sem_g, sem_d, sem_sc):
    c = lax.axis_index("c")
    s = lax.axis_index("s")
    wid = c * NS + s
    zeros16 = jnp.zeros((16,), jnp.float32)

    def zrow(r, _):
        for j in range(ROW // 16):
            outb[r, pl.ds(j * 16, 16)] = zeros16
        return _
    lax.fori_loop(0, CHUNK, zrow, None)
    row0 = s * ROWS_PER_TILE
    for k in range(ROWS_PER_TILE // CHUNK):
        pltpu.sync_copy(outb, acc.at[pl.ds(row0 + k * CHUNK, CHUNK)])
    plsc.subcore_barrier()

    iota = lax.iota(jnp.int32, 16)
    rid0 = lax.shift_right_logical(iota, 2)
    cid = lax.bitwise_and(iota, 3)

    def small(p, ci):
        base = wid * EPW + ci * CHUNK
        pltpu.async_copy(ei_hbm.at[pl.ds(base, CHUNK)], sidx[p], sem_d[p])
        pltpu.async_copy(ei_hbm.at[pl.ds(E + base, CHUNK)], didx[p],
                         sem_d[p])
        pltpu.async_copy(se_hbm.at[:, pl.ds(base, CHUNK)], seb[p], sem_d[p])

    def gathers(p, ci):
        base = wid * EPW + ci * CHUNK
        pltpu.make_async_copy(ei_hbm.at[pl.ds(base, CHUNK)], sidx[p],
                              sem_d[p]).wait()
        pltpu.make_async_copy(ei_hbm.at[pl.ds(E + base, CHUNK)], didx[p],
                              sem_d[p]).wait()
        pltpu.make_async_copy(se_hbm.at[:, pl.ds(base, CHUNK)], seb[p],
                              sem_d[p]).wait()
        pltpu.async_copy(tab_hbm.at[sidx[p]], stage[p], sem_g[p])
        pltpu.async_copy(sdst_hbm.at[didx[p]], sdstb[p], sem_g[p])

    def wait_gather(p):
        pltpu.make_async_copy(tab_hbm.at[sidx[p]], stage[p], sem_g[p]).wait()
        pltpu.make_async_copy(sdst_hbm.at[didx[p]], sdstb[p],
                              sem_g[p]).wait()

    def wait_scatter():
        pltpu.make_async_copy(outb, acc.at[dsidx], sem_sc).wait()

    def compute(p):
        stg, sdb, seb_p = stage[p], sdstb[p], seb[p]
        @plsc.parallel_loop(0, CHUNK // 4, 1, unroll=2)
        def _(v):
            rid = rid0 + (4 * v)
            s1 = plsc.load_gather(stg, [rid, cid + HF])
            s2 = plsc.load_gather(sdb, [rid, cid])
            se = plsc.load_gather(seb_p, [cid, rid])
            sc = s1 + s2 + se
            sc = jnp.maximum(sc, ALPHA * sc)
            ex = jnp.exp(sc)
            exbuf[pl.ds(16 * v, 16)] = ex
            plsc.store_scatter(outb, [rid, cid + HF], ex)

        @plsc.parallel_loop(0, CHUNK // 4, 1, unroll=4)
        def _(q):
            f = exbuf[pl.ds(16 * q, 16)]
            for r in range(4):
                e = 4 * q + r
                for hh in range(H):
                    fs = f[4 * r + hh]
                    for j in range(2):
                        sl = pl.ds(hh * F + j * 16, 16)
                        outb[e, sl] = stg[e, sl] * fs

    def scatter(p):
        for t in range(CHUNK // 16):
            dsidx[pl.ds(16 * t, 16)] = didx[p][pl.ds(16 * t, 16)]
        pltpu.async_copy(outb, acc.at[dsidx], sem_sc, add=True)

    small(0, 0)
    small(1, 1)
    gathers(0, 0)
    gathers(1, 1)
    wait_gather(0)
    compute(0)
    scatter(0)
    small(0, 2)

    def pair(k, _):
        i0 = 2 * k
        gathers(0, i0 + 2)
        wait_gather(1)
        wait_scatter()
        compute(1)
        scatter(1)
        small(1, i0 + 3)
        gathers(1, i0 + 3)
        wait_gather(0)
        wait_scatter()
        compute(0)
        scatter(0)
        small(0, i0 + 4)
        return _
    lax.fori_loop(0, (NCHUNK - 3) // 2, pair, None)

    gathers(0, 124)
    wait_gather(1)
    wait_scatter()
    compute(1)
    scatter(1)
    wait_gather(0)
    wait_scatter()
    compute(0)
    scatter(0)
    wait_scatter()

    plsc.subcore_barrier()
    for k in range(ROWS_PER_TILE // CHUNK):
        r = row0 + k * CHUNK
        pltpu.sync_copy(acc.at[pl.ds(r, CHUNK)], outb)
        pltpu.sync_copy(outb, out_hbm.at[c, pl.ds(r, CHUNK)])


def _sc_sweep(tab, sdst_tab, seT, ei_flat):
    mesh = plsc.VectorSubcoreMesh(
        core_axis_name="c", subcore_axis_name="s",
        num_cores=NC, num_subcores=NS)
    fn = pl.kernel(
        _sc_body,
        out_type=jax.ShapeDtypeStruct((NC, NPAD, ROW), jnp.float32),
        mesh=mesh,
        compiler_params=pltpu.CompilerParams(
            use_tc_tiling_on_sc=False, needs_layout_passes=False),
        scratch_types=[
            (pltpu.VMEM((CHUNK, ROW), jnp.float32),) * 2,
            pltpu.VMEM((CHUNK, ROW), jnp.float32),
            (pltpu.VMEM((CHUNK, 16), jnp.float32),) * 2,
            (pltpu.VMEM((H, CHUNK), jnp.float32),) * 2,
            (pltpu.VMEM((CHUNK,), jnp.int32),) * 2,
            (pltpu.VMEM((CHUNK,), jnp.int32),) * 2,
            pltpu.VMEM((CHUNK,), jnp.int32),
            pltpu.VMEM((CHUNK * H,), jnp.float32),
            pltpu.VMEM_SHARED((NPAD, ROW), jnp.float32),
            (pltpu.SemaphoreType.DMA,) * 2,
            (pltpu.SemaphoreType.DMA,) * 2,
            pltpu.SemaphoreType.DMA,
        ],
    )
    return fn(tab, sdst_tab, seT, ei_flat)


FBLK = 80


def _fin_body(p_hbm, out_hbm, b0, b1, ob, sem):
    c = lax.axis_index("c")
    s = lax.axis_index("s")
    wid = c * NS + s
    r0w = wid * (NPAD // NW)
    nblk = jnp.minimum(N - r0w, NPAD // NW) // FBLK

    def blk(b, _):
        r0 = r0w + b * FBLK
        c0 = pltpu.async_copy(p_hbm.at[0, pl.ds(r0, FBLK)], b0, sem)
        c1 = pltpu.async_copy(p_hbm.at[1, pl.ds(r0, FBLK)], b1, sem)
        c0.wait()
        c1.wait()

        @plsc.parallel_loop(0, FBLK, 1, unroll=2)
        def _(e):
            dv = b0[e, pl.ds(HF, 16)] + b1[e, pl.ds(HF, 16)]
            rden = 1.0 / (dv + 1e-16)
            for hh in range(H):
                rd = rden[hh]
                for j in range(2):
                    sl = pl.ds(hh * F + j * 16, 16)
                    z = (b0[e, sl] + b1[e, sl]) * rd
                    ob[e, sl] = jnp.where(
                        z > 0.0, z, jnp.exp(jnp.minimum(z, 0.0)) - 1.0)
        pltpu.sync_copy(ob, out_hbm.at[pl.ds(r0, FBLK)])
        return _
    lax.fori_loop(0, nblk, blk, None)


def _final_kernel(partials):
    mesh = plsc.VectorSubcoreMesh(
        core_axis_name="c", subcore_axis_name="s",
        num_cores=NC, num_subcores=NS)
    fn = pl.kernel(
        _fin_body,
        out_type=jax.ShapeDtypeStruct((N, HF), jnp.float32),
        mesh=mesh,
        compiler_params=pltpu.CompilerParams(
            use_tc_tiling_on_sc=False, needs_layout_passes=False),
        scratch_types=[
            pltpu.VMEM((FBLK, ROW), jnp.float32),
            pltpu.VMEM((FBLK, ROW), jnp.float32),
            pltpu.VMEM((FBLK, HF), jnp.float32),
            pltpu.SemaphoreType.DMA,
        ],
    )
    return fn(partials)


@jax.jit
def kernel(h, edge_features, edge_index, W, W_e, a_src, a_dst, a_edge):
    ar = a_src.reshape(1, HF)
    ad = a_dst.reshape(1, HF)
    ae = a_edge.reshape(1, HF)
    tab, sdst_tab = _node_kernel(h, W, ar, ad)
    seT = _edge_kernel(edge_features.T, W_e, ae)
    partials = _sc_sweep(tab, sdst_tab, seT, edge_index.reshape(2 * E))
    return _final_kernel(partials)


# speedup vs baseline: 1.1257x; 1.0012x over previous
import jax
import jax.numpy as jnp
from jax import lax
from jax.experimental import pallas as pl
from jax.experimental.pallas import tpu as pltpu
from jax.experimental.pallas import tpu_sc as plsc

N = 10000
E = 320000
H = 4
F = 32
HF = H * F
ROW = HF + 16
ALPHA = 0.2

NC = 2
NS = 16
NW = NC * NS
EPW = E // NW
CHUNK = 80
NCHUNK = EPW // CHUNK
NPAD = 10240
ROWS_PER_TILE = NPAD // NS


def _node_body(h_ref, w_ref, ar_ref, ad_ref, tab_ref, sd_ref):
    w = w_ref[...]
    hb = h_ref[...]
    wh = jnp.dot(hb, w, preferred_element_type=jnp.float32)
    wm_s = w * ar_ref[...]
    wm_d = w * ad_ref[...]
    cols = []
    for hh in range(H):
        cols.append(jnp.sum(wm_s[:, hh * F:(hh + 1) * F], axis=1, keepdims=True))
    for hh in range(H):
        cols.append(jnp.sum(wm_d[:, hh * F:(hh + 1) * F], axis=1, keepdims=True))
    csd = jnp.concatenate(cols + [jnp.zeros((HF, 120), jnp.float32)], axis=1)
    s = jnp.dot(hb, csd, preferred_element_type=jnp.float32)
    bn = hb.shape[0]
    tab_ref[...] = jnp.concatenate(
        [wh, s[:, 0:4], jnp.zeros((bn, 12), jnp.float32)], axis=1)
    sd_ref[...] = jnp.concatenate(
        [s[:, 4:8], jnp.zeros((bn, 12), jnp.float32)], axis=1)


def _node_kernel(h, W, ar, ad):
    BN = 1000
    grid = (N // BN,)
    return pl.pallas_call(
        _node_body,
        grid=grid,
        in_specs=[
            pl.BlockSpec((BN, HF), lambda i: (i, 0)),
            pl.BlockSpec((HF, HF), lambda i: (0, 0)),
            pl.BlockSpec((1, HF), lambda i: (0, 0)),
            pl.BlockSpec((1, HF), lambda i: (0, 0)),
        ],
        out_specs=[
            pl.BlockSpec((BN, ROW), lambda i: (i, 0)),
            pl.BlockSpec((BN, 16), lambda i: (i, 0)),
        ],
        out_shape=[
            jax.ShapeDtypeStruct((N, ROW), jnp.float32),
            jax.ShapeDtypeStruct((N, 16), jnp.float32),
        ],
    )(h, W, ar, ad)


def _edge_body(eft_ref, we_ref, ae_ref, out_ref):
    we = we_ref[...]
    wm = we * ae_ref[...]
    cols = []
    for hh in range(H):
        cols.append(jnp.sum(wm[:, hh * F:(hh + 1) * F], axis=1, keepdims=True))
    ce = jnp.concatenate(cols, axis=1)
    out_ref[...] = lax.dot_general(
        ce, eft_ref[...], (((0,), (0,)), ((), ())),
        preferred_element_type=jnp.float32)


def _edge_kernel(eft, W_e, ae):
    BE = 12800
    EF = eft.shape[0]
    grid = (E // BE,)
    return pl.pallas_call(
        _edge_body,
        grid=grid,
        in_specs=[
            pl.BlockSpec((EF, BE), lambda i: (0, i)),
            pl.BlockSpec((EF, HF), lambda i: (0, 0)),
            pl.BlockSpec((1, HF), lambda i: (0, 0)),
        ],
        out_specs=pl.BlockSpec((H, BE), lambda i: (0, i)),
        out_shape=jax.ShapeDtypeStruct((H, E), jnp.float32),
    )(eft, W_e, ae)


def _sc_body(tab_hbm, sdst_hbm, se_hbm, ei_hbm, out_hbm,
             stage, outb, sdstb, seb, sidx, didx, dsidx, exbuf,
             acc, sem_g, sem_d, sem_sc):
    c = lax.axis_index("c")
    s = lax.axis_index("s")
    wid = c * NS + s
    zeros16 = jnp.zeros((16,), jnp.float32)

    def zrow(r, _):
        for j in range(ROW // 16):
            outb[r, pl.ds(j * 16, 16)] = zeros16
        return _
    lax.fori_loop(0, CHUNK, zrow, None)
    row0 = s * ROWS_PER_TILE
    for k in range(ROWS_PER_TILE // CHUNK):
        pltpu.sync_copy(outb, acc.at[pl.ds(row0 + k * CHUNK, CHUNK)])
    plsc.subcore_barrier()

    iota = lax.iota(jnp.int32, 16)
    rid0 = lax.shift_right_logical(iota, 2)
    cid = lax.bitwise_and(iota, 3)

    def small(p, ci):
        base = wid * EPW + ci * CHUNK
        pltpu.async_copy(ei_hbm.at[pl.ds(base, CHUNK)], sidx[p], sem_d[p])
        pltpu.async_copy(ei_hbm.at[pl.ds(E + base, CHUNK)], didx[p],
                         sem_d[p])
        pltpu.async_copy(se_hbm.at[:, pl.ds(base, CHUNK)], seb[p], sem_d[p])

    def gathers(p, ci):
        base = wid * EPW + ci * CHUNK
        pltpu.make_async_copy(ei_hbm.at[pl.ds(base, CHUNK)], sidx[p],
                              sem_d[p]).wait()
        pltpu.make_async_copy(ei_hbm.at[pl.ds(E + base, CHUNK)], didx[p],
                              sem_d[p]).wait()
        pltpu.make_async_copy(se_hbm.at[:, pl.ds(base, CHUNK)], seb[p],
                              sem_d[p]).wait()
        pltpu.async_copy(tab_hbm.at[sidx[p]], stage[p], sem_g[p])
        pltpu.async_copy(sdst_hbm.at[didx[p]], sdstb[p], sem_g[p])

    def wait_gather(p):
        pltpu.make_async_copy(tab_hbm.at[sidx[p]], stage[p], sem_g[p]).wait()
        pltpu.make_async_copy(sdst_hbm.at[didx[p]], sdstb[p],
                              sem_g[p]).wait()

    def wait_scatter():
        pltpu.make_async_copy(outb, acc.at[dsidx], sem_sc).wait()

    def compute(p):
        stg, sdb, seb_p = stage[p], sdstb[p], seb[p]
        @plsc.parallel_loop(0, CHUNK // 4, 1, unroll=2)
        def _(v):
            rid = rid0 + (4 * v)
            s1 = plsc.load_gather(stg, [rid, cid + HF])
            s2 = plsc.load_gather(sdb, [rid, cid])
            se = plsc.load_gather(seb_p, [cid, rid])
            sc = s1 + s2 + se
            sc = jnp.maximum(sc, ALPHA * sc)
            ex = jnp.exp(sc)
            exbuf[pl.ds(16 * v, 16)] = ex
            plsc.store_scatter(outb, [rid, cid + HF], ex)

        @plsc.parallel_loop(0, CHUNK // 4, 1, unroll=4)
        def _(q):
            f = exbuf[pl.ds(16 * q, 16)]
            for r in range(4):
                e = 4 * q + r
                for hh in range(H):
                    fs = f[4 * r + hh]
                    for j in range(2):
                        sl = pl.ds(hh * F + j * 16, 16)
                        outb[e, sl] = stg[e, sl] * fs

    def scatter(p):
        for t in range(CHUNK // 16):
            dsidx[pl.ds(16 * t, 16)] = didx[p][pl.ds(16 * t, 16)]
        pltpu.async_copy(outb, acc.at[dsidx], sem_sc, add=True)

    small(0, 0)
    small(1, 1)
    gathers(0, 0)
    gathers(1, 1)
    wait_gather(0)
    compute(0)
    scatter(0)
    small(0, 2)

    def pair(k, _):
        i0 = 2 * k
        gathers(0, i0 + 2)
        wait_gather(1)
        wait_scatter()
        compute(1)
        scatter(1)
        small(1, i0 + 3)
        gathers(1, i0 + 3)
        wait_gather(0)
        wait_scatter()
        compute(0)
        scatter(0)
        small(0, i0 + 4)
        return _
    lax.fori_loop(0, (NCHUNK - 3) // 2, pair, None)

    gathers(0, 124)
    wait_gather(1)
    wait_scatter()
    compute(1)
    scatter(1)
    wait_gather(0)
    wait_scatter()
    compute(0)
    scatter(0)
    wait_scatter()

    plsc.subcore_barrier()
    for k in range(ROWS_PER_TILE // CHUNK):
        r = row0 + k * CHUNK
        pltpu.sync_copy(acc.at[pl.ds(r, CHUNK)], outb)
        pltpu.sync_copy(outb, out_hbm.at[c, pl.ds(r, CHUNK)])


def _sc_sweep(tab, sdst_tab, seT, ei_flat):
    mesh = plsc.VectorSubcoreMesh(
        core_axis_name="c", subcore_axis_name="s",
        num_cores=NC, num_subcores=NS)
    fn = pl.kernel(
        _sc_body,
        out_type=jax.ShapeDtypeStruct((NC, NPAD, ROW), jnp.float32),
        mesh=mesh,
        compiler_params=pltpu.CompilerParams(
            use_tc_tiling_on_sc=False, needs_layout_passes=False),
        scratch_types=[
            (pltpu.VMEM((CHUNK, ROW), jnp.float32),) * 2,
            pltpu.VMEM((CHUNK, ROW), jnp.float32),
            (pltpu.VMEM((CHUNK, 16), jnp.float32),) * 2,
            (pltpu.VMEM((H, CHUNK), jnp.float32),) * 2,
            (pltpu.VMEM((CHUNK,), jnp.int32),) * 2,
            (pltpu.VMEM((CHUNK,), jnp.int32),) * 2,
            pltpu.VMEM((CHUNK,), jnp.int32),
            pltpu.VMEM((CHUNK * H,), jnp.float32),
            pltpu.VMEM_SHARED((NPAD, ROW), jnp.float32),
            (pltpu.SemaphoreType.DMA,) * 2,
            (pltpu.SemaphoreType.DMA,) * 2,
            pltpu.SemaphoreType.DMA,
        ],
    )
    return fn(tab, sdst_tab, seT, ei_flat)


FBLK = 80


def _fin_body(p_hbm, out_hbm, b0, b1, ob, sem):
    c = lax.axis_index("c")
    s = lax.axis_index("s")
    wid = c * NS + s
    r0w = wid * (NPAD // NW)
    nblk = jnp.minimum(N - r0w, NPAD // NW) // FBLK

    def blk(b, _):
        r0 = r0w + b * FBLK
        c0 = pltpu.async_copy(p_hbm.at[0, pl.ds(r0, FBLK)], b0, sem)
        c1 = pltpu.async_copy(p_hbm.at[1, pl.ds(r0, FBLK)], b1, sem)
        c0.wait()
        c1.wait()

        @plsc.parallel_loop(0, FBLK, 1, unroll=2)
        def _(e):
            dv = b0[e, pl.ds(HF, 16)] + b1[e, pl.ds(HF, 16)]
            rden = 1.0 / (dv + 1e-16)
            for hh in range(H):
                rd = rden[hh]
                for j in range(2):
                    sl = pl.ds(hh * F + j * 16, 16)
                    z = (b0[e, sl] + b1[e, sl]) * rd
                    ob[e, sl] = jnp.where(
                        z > 0.0, z, jnp.exp(jnp.minimum(z, 0.0)) - 1.0)
        pltpu.sync_copy(ob, out_hbm.at[pl.ds(r0, FBLK)])
        return _
    lax.fori_loop(0, nblk, blk, None)


def _final_kernel(partials):
    mesh = plsc.VectorSubcoreMesh(
        core_axis_name="c", subcore_axis_name="s",
        num_cores=NC, num_subcores=NS)
    fn = pl.kernel(
        _fin_body,
        out_type=jax.ShapeDtypeStruct((N, HF), jnp.float32),
        mesh=mesh,
        compiler_params=pltpu.CompilerParams(
            use_tc_tiling_on_sc=False, needs_layout_passes=False),
        scratch_types=[
            pltpu.VMEM((FBLK, ROW), jnp.float32),
            pltpu.VMEM((FBLK, ROW), jnp.float32),
            pltpu.VMEM((FBLK, HF), jnp.float32),
            pltpu.SemaphoreType.DMA,
        ],
    )
    return fn(partials)


@jax.jit
def kernel(h, edge_features, edge_index, W, W_e, a_src, a_dst, a_edge):
    ar = a_src.reshape(1, HF)
    ad = a_dst.reshape(1, HF)
    ae = a_edge.reshape(1, HF)
    tab, sdst_tab = _node_kernel(h, W, ar, ad)
    seT = _edge_kernel(edge_features.T, W_e, ae)
    partials = _sc_sweep(tab, sdst_tab, seT, edge_index.reshape(2 * E))
    return _final_kernel(partials)
